# baseline (speedup 1.0000x reference)
import functools

import jax
import jax.numpy as jnp
from jax import lax
from jax.experimental import pallas as pl
from jax.experimental.pallas import tpu as pltpu
from jax.experimental.pallas import tpu_sc as plsc

N = 4096
M = 62
D = 128
C = 64
NEG = -1e9

R_BLK = 512


def _comp_body(u_ref, w_ref, src_ref, dst_ref,
               rest_ref, s63_ref, d2d_ref, dc_ref, sh_ref):
    f32 = jnp.float32
    zrow = jnp.zeros((1, D), f32)
    Ue = jnp.concatenate([zrow, u_ref[0:M], zrow], axis=0)
    W = w_ref[:]
    src = src_ref[:]
    dst = dst_ref[:]

    def mm(a, b):
        return lax.dot_general(a, b, (((1,), (0,)), ((), ())),
                               preferred_element_type=f32)

    def mmt(a, b):
        return lax.dot_general(a, b, (((1,), (1,)), ((), ())),
                               preferred_element_type=f32)

    QUe = mm(Ue, W)
    Qsrc = mm(src, W)
    Qdst = mm(dst, W)

    col = lax.broadcasted_iota(jnp.int32, (R_BLK, C), 1)
    is63 = col == C - 1

    ndst = jnp.sum(dst * dst, axis=1, keepdims=True)
    nsrc = jnp.sum(src * src, axis=1, keepdims=True)
    nU = jnp.sum(Ue * Ue, axis=1)

    is0 = col == 0
    S0 = mmt(Qsrc, Ue)
    S63 = mmt(Qdst, Ue)
    S63 = jnp.where(is63, jnp.sum(Qdst * dst, axis=1, keepdims=True), S63)
    S63 = jnp.where(is0, NEG, S63)
    DC = mmt(dst, QUe)
    D2dst = ndst + nU[None, :] - 2.0 * mmt(dst, Ue)
    D2dst = jnp.where(is63, 0.0, D2dst)
    D2src = nsrc + nU[None, :] - 2.0 * mmt(src, Ue)

    rest_ref[:] = jnp.concatenate([S0, D2src], axis=1)
    s63_ref[:] = S63
    d2d_ref[:] = D2dst
    dc_ref[:] = DC
    colq = lax.broadcasted_iota(jnp.int32, (C, C), 1)
    SUU = jnp.where(colq == 0, NEG, mmt(QUe, Ue))
    D2UU = nU[:, None] + nU[None, :] - 2.0 * mmt(Ue, Ue)
    sh_ref[:] = jnp.concatenate([SUU, D2UU], axis=0)


_components = functools.partial(
    pl.pallas_call,
    _comp_body,
    grid=(N // R_BLK,),
    in_specs=[
        pl.BlockSpec((C, D), lambda i: (2 * N // C, 0)),
        pl.BlockSpec((D, D), lambda i: (0, 0)),
        pl.BlockSpec((R_BLK, D), lambda i: (i, 0)),
        pl.BlockSpec((R_BLK, D), lambda i: (i + N // R_BLK, 0)),
    ],
    out_specs=[
        pl.BlockSpec((R_BLK, 2 * C), lambda i: (i, 0)),
        pl.BlockSpec((R_BLK, C), lambda i: (i, 0)),
        pl.BlockSpec((R_BLK, C), lambda i: (i, 0)),
        pl.BlockSpec((R_BLK, C), lambda i: (i, 0)),
        pl.BlockSpec((2 * C, C), lambda i: (0, 0)),
    ],
    out_shape=[
        jax.ShapeDtypeStruct((N, 2 * C), jnp.float32),
        jax.ShapeDtypeStruct((N, C), jnp.float32),
        jax.ShapeDtypeStruct((N, C), jnp.float32),
        jax.ShapeDtypeStruct((N, C), jnp.float32),
        jax.ShapeDtypeStruct((2 * C, C), jnp.float32),
    ],
)


NUM_WORKERS = 32
ROWS_PER_W = N // NUM_WORKERS
GROUPS = ROWS_PER_W // 16
IL = 2
REST_STRIDE = 2 * C + 1
TBL_STRIDE = C + 1
T_S63, T_D2UU, T_D2D, T_DC = 64, 192, 256, 384
B_D2S = 64


def _decode_body(rest_hbm, s63_hbm, d2d_hbm, dc_hbm, sh_hbm, out_hbm,
                 rest_v, tbl_v, cid_v, pos_v, out_v):
    wid = lax.axis_index("s") * 2 + lax.axis_index("c")
    base = wid * ROWS_PER_W
    rows = pl.ds(base, ROWS_PER_W)
    cc = pl.ds(0, C)
    pltpu.sync_copy(rest_hbm.at[rows], rest_v.at[:, pl.ds(0, 2 * C)])
    pltpu.sync_copy(sh_hbm.at[pl.ds(0, C)], tbl_v.at[pl.ds(0, C), cc])
    pltpu.sync_copy(s63_hbm.at[rows], tbl_v.at[pl.ds(T_S63, ROWS_PER_W), cc])
    pltpu.sync_copy(sh_hbm.at[pl.ds(C, C)], tbl_v.at[pl.ds(T_D2UU, C), cc])
    pltpu.sync_copy(d2d_hbm.at[rows], tbl_v.at[pl.ds(T_D2D, ROWS_PER_W), cc])
    pltpu.sync_copy(dc_hbm.at[rows], tbl_v.at[pl.ds(T_DC, ROWS_PER_W), cc])

    lanes = lax.iota(jnp.int32, 16)
    zero = jnp.zeros((16,), jnp.int32)
    negv = jnp.full((16,), NEG, jnp.float32)
    zf = jnp.zeros((16,), jnp.float32)
    ninf = jnp.full((16,), -jnp.inf, jnp.float32)

    idc = [jnp.where(lanes + 16 * c + 1 <= M, lanes + 16 * c + 1, 0)
           for c in range(C // 16)]
    psc = [lanes + 16 * c - 1 for c in range(C // 16)]

    for gp in range(0, GROUPS, IL):
        rv = [lanes + (gp + i) * 16 for i in range(IL)]
        ml = [lanes + i * 16 for i in range(IL)]

        for r in range(16 * IL):
            for c4 in range(C // 16):
                cid_v[r, pl.ds(c4 * 16, 16)] = idc[c4]
                pos_v[r, pl.ds(c4 * 16, 16)] = psc[c4]

        def s0_body(s, carry):
            bests, bestis = carry
            sv = zero + s
            nb, ni = [], []
            for i in range(IL):
                v = plsc.load_gather(rest_v, [rv[i], sv])
                gt = v > bests[i]
                nb.append(jnp.where(gt, v, bests[i]))
                ni.append(jnp.where(gt, sv, bestis[i]))
            return tuple(nb), tuple(ni)

        _, sps = lax.fori_loop(1, 63, s0_body,
                               ((ninf,) * IL, (zero,) * IL), unroll=8)
        mds = []
        for i in range(IL):
            d2 = plsc.load_gather(rest_v, [rv[i], sps[i] + B_D2S])
            mds.append(jnp.maximum(d2, 0.0))
            k = plsc.load_gather(pos_v, [ml[i], sps[i]])
            last = zero + (M - 1)
            lastval = plsc.load_gather(cid_v, [ml[i], last])
            plsc.store_scatter(cid_v, [ml[i], k], lastval)
            plsc.store_scatter(cid_v, [ml[i], last], zero)
            plsc.store_scatter(pos_v, [ml[i], lastval], k)
        cnts = (zero + (M - 1),) * IL

        def step(_, carry):
            js, cnts, md2s = carry
            ps, v63s = [], []
            for i in range(IL):
                j = js[i]
                isD = j == C - 1
                ps.append(jnp.where(isD, T_S63 + rv[i], j))
                q = jnp.where(isD, T_S63 + rv[i], T_DC + rv[i])
                c63 = jnp.where(isD, zero + (C - 1), j)
                v63s.append(plsc.load_gather(tbl_v, [q, c63]))

            bound = cnts[0]
            for i in range(1, IL):
                bound = jnp.maximum(bound, cnts[i])
            bound = jnp.max(bound)

            def wcond(c):
                return c[0] < bound

            def wbody(c):
                t, bests, bestis = c
                nb, ni = list(bests), list(bestis)
                for u in range(16):
                    sv = zero + (t + u)
                    for i in range(IL):
                        cid = plsc.load_gather(cid_v, [ml[i], sv])
                        v = plsc.load_gather(tbl_v, [ps[i], cid])
                        gt = v > nb[i]
                        nb[i] = jnp.where(gt, v, nb[i])
                        ni[i] = jnp.where(gt, cid, ni[i])
                return t + 16, tuple(nb), tuple(ni)

            _, bests, bestis = lax.while_loop(
                wcond, wbody, (jnp.int32(0), (ninf,) * IL, (zero,) * IL))

            njs, ncnts, nmds = [], [], []
            for i in range(IL):
                j = js[i]
                isD = j == C - 1
                gt = v63s[i] > bests[i]
                sp = jnp.where(gt, zero + (C - 1), bestis[i])
                sp63 = sp == C - 1
                row2 = jnp.where(isD | sp63, T_D2D + rv[i], T_D2UU + j)
                col2 = jnp.where(sp63 & (~isD), j, sp)
                d2 = plsc.load_gather(tbl_v, [row2, col2])
                nmds.append(jnp.maximum(md2s[i], jnp.maximum(d2, 0.0)))
                cnt1 = cnts[i] - jnp.where(sp63, 0, 1)
                k = plsc.load_gather(pos_v, [ml[i], sp])
                lastval = plsc.load_gather(cid_v, [ml[i], cnt1])
                kk = jnp.where(sp63, zero + C, k)
                ls = jnp.where(sp63, zero + C, cnt1)
                plsc.store_scatter(cid_v, [ml[i], kk], lastval)
                plsc.store_scatter(cid_v, [ml[i], ls], zero)
                plsc.store_scatter(pos_v, [ml[i], jnp.where(sp63, zero,
                                                            lastval)], k)
                njs.append(sp)
                ncnts.append(cnt1)
            return tuple(njs), tuple(ncnts), tuple(nmds)

        _, _, mds = lax.fori_loop(1, C, step, (tuple(sps), cnts, tuple(mds)))
        for i in range(IL):
            out_v[pl.ds((gp + i) * 16, 16)] = mds[i]

    pltpu.sync_copy(out_v, out_hbm.at[wid])


@functools.cache
def _decode():
    mesh = plsc.VectorSubcoreMesh(core_axis_name="c", subcore_axis_name="s",
                                  num_cores=2, num_subcores=16)
    return pl.kernel(
        _decode_body,
        out_type=jax.ShapeDtypeStruct((NUM_WORKERS, ROWS_PER_W), jnp.float32),
        mesh=mesh,
        scratch_types=[
            pltpu.VMEM((ROWS_PER_W, REST_STRIDE), jnp.float32),
            pltpu.VMEM((8 * C, TBL_STRIDE), jnp.float32),
            pltpu.VMEM((16 * IL, TBL_STRIDE), jnp.int32),
            pltpu.VMEM((16 * IL, TBL_STRIDE), jnp.int32),
            pltpu.VMEM((ROWS_PER_W,), jnp.float32),
        ],
        compiler_params=pltpu.CompilerParams(use_tc_tiling_on_sc=False,
                                             needs_layout_passes=False),
    )


def _final_body(x_ref, o_ref):
    o_ref[0, 0] = jnp.sum(jnp.sqrt(x_ref[:])) * (1.0 / N)


_finalize = functools.partial(
    pl.pallas_call,
    _final_body,
    in_specs=[pl.BlockSpec((NUM_WORKERS, ROWS_PER_W), lambda: (0, 0))],
    out_specs=pl.BlockSpec(memory_space=pltpu.SMEM),
    out_shape=jax.ShapeDtypeStruct((1, 1), jnp.float32),
)


def kernel(outputs, W, Wh):
    del Wh
    rest, s63, d2d, dc, sh = _components()(outputs, W, outputs, outputs)
    maxd2 = _decode()(rest, s63, d2d, dc, sh)
    res = _finalize()(maxd2)
    return res[0, 0]


# speedup vs baseline: 1.2422x; 1.2422x over previous
import functools

import jax
import jax.numpy as jnp
from jax import lax
from jax.experimental import pallas as pl
from jax.experimental.pallas import tpu as pltpu
from jax.experimental.pallas import tpu_sc as plsc

N = 4096
M = 62
D = 128
C = 64
NEG = -1e9

R_BLK = 512


def _comp_body(u_ref, w_ref, src_ref, dst_ref,
               rest_ref, s63_ref, d2d_ref, dc_ref, sh_ref):
    f32 = jnp.float32
    zrow = jnp.zeros((1, D), f32)
    Ue = jnp.concatenate([zrow, u_ref[0:M], zrow], axis=0)
    W = w_ref[:]
    src = src_ref[:]
    dst = dst_ref[:]

    def mm(a, b):
        return lax.dot_general(a, b, (((1,), (0,)), ((), ())),
                               preferred_element_type=f32)

    def mmt(a, b):
        return lax.dot_general(a, b, (((1,), (1,)), ((), ())),
                               preferred_element_type=f32)

    QUe = mm(Ue, W)
    Qsrc = mm(src, W)
    Qdst = mm(dst, W)

    col = lax.broadcasted_iota(jnp.int32, (R_BLK, C), 1)
    is63 = col == C - 1

    ndst = jnp.sum(dst * dst, axis=1, keepdims=True)
    nsrc = jnp.sum(src * src, axis=1, keepdims=True)
    nU = jnp.sum(Ue * Ue, axis=1)

    is0 = col == 0
    S0 = mmt(Qsrc, Ue)
    S63 = mmt(Qdst, Ue)
    S63 = jnp.where(is63, jnp.sum(Qdst * dst, axis=1, keepdims=True), S63)
    S63 = jnp.where(is0, NEG, S63)
    DC = mmt(dst, QUe)
    D2dst = ndst + nU[None, :] - 2.0 * mmt(dst, Ue)
    D2dst = jnp.where(is63, 0.0, D2dst)
    D2src = nsrc + nU[None, :] - 2.0 * mmt(src, Ue)

    rest_ref[:] = jnp.concatenate([S0, D2src], axis=1)
    s63_ref[:] = S63
    d2d_ref[:] = D2dst
    dc_ref[:] = DC
    colq = lax.broadcasted_iota(jnp.int32, (C, C), 1)
    SUU = jnp.where(colq == 0, NEG, mmt(QUe, Ue))
    D2UU = nU[:, None] + nU[None, :] - 2.0 * mmt(Ue, Ue)
    sh_ref[:] = jnp.concatenate([SUU, D2UU], axis=0)


_components = functools.partial(
    pl.pallas_call,
    _comp_body,
    grid=(N // R_BLK,),
    in_specs=[
        pl.BlockSpec((C, D), lambda i: (2 * N // C, 0)),
        pl.BlockSpec((D, D), lambda i: (0, 0)),
        pl.BlockSpec((R_BLK, D), lambda i: (i, 0)),
        pl.BlockSpec((R_BLK, D), lambda i: (i + N // R_BLK, 0)),
    ],
    out_specs=[
        pl.BlockSpec((R_BLK, 2 * C), lambda i: (i, 0)),
        pl.BlockSpec((R_BLK, C), lambda i: (i, 0)),
        pl.BlockSpec((R_BLK, C), lambda i: (i, 0)),
        pl.BlockSpec((R_BLK, C), lambda i: (i, 0)),
        pl.BlockSpec((2 * C, C), lambda i: (0, 0)),
    ],
    out_shape=[
        jax.ShapeDtypeStruct((N, 2 * C), jnp.float32),
        jax.ShapeDtypeStruct((N, C), jnp.float32),
        jax.ShapeDtypeStruct((N, C), jnp.float32),
        jax.ShapeDtypeStruct((N, C), jnp.float32),
        jax.ShapeDtypeStruct((2 * C, C), jnp.float32),
    ],
)


NUM_WORKERS = 32
ROWS_PER_W = N // NUM_WORKERS
GROUPS = ROWS_PER_W // 16
IL = 2
REST_STRIDE = 2 * C + 1
TBL_STRIDE = C + 1
T_S63, T_D2UU, T_D2D, T_DC = 64, 192, 256, 384
B_D2S = 64


def _decode_body(rest_hbm, s63_hbm, d2d_hbm, dc_hbm, sh_hbm, out_hbm,
                 rest_v, tbl_v, cid_v, pos_v, out_v, sem):
    wid = lax.axis_index("s") * 2 + lax.axis_index("c")
    base = wid * ROWS_PER_W
    rows = pl.ds(base, ROWS_PER_W)
    cc = pl.ds(0, C)
    cps = [
        pltpu.async_copy(rest_hbm.at[rows], rest_v.at[:, pl.ds(0, 2 * C)],
                         sem),
        pltpu.async_copy(sh_hbm.at[pl.ds(0, C)], tbl_v.at[pl.ds(0, C), cc],
                         sem),
        pltpu.async_copy(s63_hbm.at[rows],
                         tbl_v.at[pl.ds(T_S63, ROWS_PER_W), cc], sem),
        pltpu.async_copy(sh_hbm.at[pl.ds(C, C)],
                         tbl_v.at[pl.ds(T_D2UU, C), cc], sem),
        pltpu.async_copy(d2d_hbm.at[rows],
                         tbl_v.at[pl.ds(T_D2D, ROWS_PER_W), cc], sem),
        pltpu.async_copy(dc_hbm.at[rows],
                         tbl_v.at[pl.ds(T_DC, ROWS_PER_W), cc], sem),
    ]
    for cp in cps:
        cp.wait()

    lanes = lax.iota(jnp.int32, 16)
    zero = jnp.zeros((16,), jnp.int32)
    negv = jnp.full((16,), NEG, jnp.float32)
    zf = jnp.zeros((16,), jnp.float32)
    ninf = jnp.full((16,), -jnp.inf, jnp.float32)

    idc = [jnp.where(lanes + 16 * c + 1 <= M, lanes + 16 * c + 1, 0)
           for c in range(C // 16)]
    psc = [lanes + 16 * c - 1 for c in range(C // 16)]

    for gp in range(0, GROUPS, IL):
        rv = [lanes + (gp + i) * 16 for i in range(IL)]
        ml = [lanes + i * 16 for i in range(IL)]

        for r in range(16 * IL):
            for c4 in range(C // 16):
                cid_v[r, pl.ds(c4 * 16, 16)] = idc[c4]
                pos_v[r, pl.ds(c4 * 16, 16)] = psc[c4]

        def s0_body(s, carry):
            bests, bestis = carry
            sv = zero + s
            nb, ni = [], []
            for i in range(IL):
                v = plsc.load_gather(rest_v, [rv[i], sv])
                gt = v > bests[i]
                nb.append(jnp.where(gt, v, bests[i]))
                ni.append(jnp.where(gt, sv, bestis[i]))
            return tuple(nb), tuple(ni)

        _, sps = lax.fori_loop(1, 63, s0_body,
                               ((ninf,) * IL, (zero,) * IL), unroll=8)
        mds = []
        for i in range(IL):
            d2 = plsc.load_gather(rest_v, [rv[i], sps[i] + B_D2S])
            mds.append(jnp.maximum(d2, 0.0))
            k = plsc.load_gather(pos_v, [ml[i], sps[i]])
            last = zero + (M - 1)
            lastval = plsc.load_gather(cid_v, [ml[i], last])
            plsc.store_scatter(cid_v, [ml[i], k], lastval)
            plsc.store_scatter(cid_v, [ml[i], last], zero)
            plsc.store_scatter(pos_v, [ml[i], lastval], k)
        cnts = (zero + (M - 1),) * IL

        def step(_, carry):
            js, cnts, md2s = carry
            ps, v63s = [], []
            for i in range(IL):
                j = js[i]
                isD = j == C - 1
                ps.append(jnp.where(isD, T_S63 + rv[i], j))
                q = jnp.where(isD, T_S63 + rv[i], T_DC + rv[i])
                c63 = jnp.where(isD, zero + (C - 1), j)
                v63s.append(plsc.load_gather(tbl_v, [q, c63]))

            bound = cnts[0]
            for i in range(1, IL):
                bound = jnp.maximum(bound, cnts[i])
            bound = jnp.max(bound)

            def wcond(c):
                return c[0] < bound

            def wbody(c):
                t, bests, bestis = c
                nb, ni = list(bests), list(bestis)
                for u in range(8):
                    sv = zero + (t + u)
                    for i in range(IL):
                        cid = plsc.load_gather(cid_v, [ml[i], sv])
                        v = plsc.load_gather(tbl_v, [ps[i], cid])
                        gt = v > nb[i]
                        nb[i] = jnp.where(gt, v, nb[i])
                        ni[i] = jnp.where(gt, cid, ni[i])
                return t + 8, tuple(nb), tuple(ni)

            _, bests, bestis = lax.while_loop(
                wcond, wbody, (jnp.int32(0), (ninf,) * IL, (zero,) * IL))

            njs, ncnts, nmds = [], [], []
            for i in range(IL):
                j = js[i]
                isD = j == C - 1
                gt = v63s[i] > bests[i]
                sp = jnp.where(gt, zero + (C - 1), bestis[i])
                sp63 = sp == C - 1
                row2 = jnp.where(isD | sp63, T_D2D + rv[i], T_D2UU + j)
                col2 = jnp.where(sp63 & (~isD), j, sp)
                d2 = plsc.load_gather(tbl_v, [row2, col2])
                nmds.append(jnp.maximum(md2s[i], jnp.maximum(d2, 0.0)))
                cnt1 = cnts[i] - jnp.where(sp63, 0, 1)
                k = plsc.load_gather(pos_v, [ml[i], sp])
                lastval = plsc.load_gather(cid_v, [ml[i], cnt1])
                kk = jnp.where(sp63, zero + C, k)
                ls = jnp.where(sp63, zero + C, cnt1)
                plsc.store_scatter(cid_v, [ml[i], kk], lastval)
                plsc.store_scatter(cid_v, [ml[i], ls], zero)
                plsc.store_scatter(pos_v, [ml[i], jnp.where(sp63, zero,
                                                            lastval)], k)
                njs.append(sp)
                ncnts.append(cnt1)
            return tuple(njs), tuple(ncnts), tuple(nmds)

        _, _, mds = lax.fori_loop(1, C, step, (tuple(sps), cnts, tuple(mds)))
        for i in range(IL):
            out_v[pl.ds((gp + i) * 16, 16)] = mds[i]

    pltpu.sync_copy(out_v, out_hbm.at[wid])


@functools.cache
def _decode():
    mesh = plsc.VectorSubcoreMesh(core_axis_name="c", subcore_axis_name="s",
                                  num_cores=2, num_subcores=16)
    return pl.kernel(
        _decode_body,
        out_type=jax.ShapeDtypeStruct((NUM_WORKERS, ROWS_PER_W), jnp.float32),
        mesh=mesh,
        scratch_types=[
            pltpu.VMEM((ROWS_PER_W, REST_STRIDE), jnp.float32),
            pltpu.VMEM((8 * C, TBL_STRIDE), jnp.float32),
            pltpu.VMEM((16 * IL, TBL_STRIDE), jnp.int32),
            pltpu.VMEM((16 * IL, TBL_STRIDE), jnp.int32),
            pltpu.VMEM((ROWS_PER_W,), jnp.float32),
            pltpu.SemaphoreType.DMA,
        ],
        compiler_params=pltpu.CompilerParams(use_tc_tiling_on_sc=False,
                                             needs_layout_passes=False),
    )


def _final_body(x_ref, o_ref):
    o_ref[0, 0] = jnp.sum(jnp.sqrt(x_ref[:])) * (1.0 / N)


_finalize = functools.partial(
    pl.pallas_call,
    _final_body,
    in_specs=[pl.BlockSpec((NUM_WORKERS, ROWS_PER_W), lambda: (0, 0))],
    out_specs=pl.BlockSpec(memory_space=pltpu.SMEM),
    out_shape=jax.ShapeDtypeStruct((1, 1), jnp.float32),
)


def kernel(outputs, W, Wh):
    del Wh
    rest, s63, d2d, dc, sh = _components()(outputs, W, outputs, outputs)
    maxd2 = _decode()(rest, s63, d2d, dc, sh)
    res = _finalize()(maxd2)
    return res[0, 0]
